# baseline (speedup 1.0000x reference)
import functools

import jax
import jax.numpy as jnp
from jax import lax
from jax.experimental import pallas as pl
from jax.experimental.pallas import tpu as pltpu
from jax.experimental.pallas import tpu_sc as plsc

L = 200
C = 64


def _sc_block_gather(starts, table_flat, edge_flat, batch, p):
    num_workers = 32
    per_w = batch // num_workers
    blk = L * C
    nbuf = 4
    wrap_lim = p - L
    LF = L + 8
    mesh = plsc.VectorSubcoreMesh(core_axis_name="c", subcore_axis_name="s")

    @functools.partial(
        pl.kernel,
        out_type=jax.ShapeDtypeStruct((batch, L, C), jnp.float32),
        mesh=mesh,
        scratch_types=[
            pltpu.VMEM((per_w,), jnp.int32),
            [pltpu.VMEM((LF, C), jnp.float32) for _ in range(nbuf)],
            pltpu.SemaphoreType.DMA((nbuf,)),
            pltpu.SemaphoreType.DMA((nbuf,)),
        ],
    )
    def k(starts_hbm, table_hbm, edge_hbm, out_hbm, idx_v, bufs, in_sems, out_sems):
        wid = lax.axis_index("s") * 2 + lax.axis_index("c")
        base = wid * per_w
        pltpu.sync_copy(starts_hbm.at[pl.ds(base, per_w)], idx_v)
        svecs = [idx_v[pl.ds(g * 16, 16)] for g in range(per_w // 16)]

        def start_of(i):
            return svecs[i // 16][i % 16]

        def gather(i):
            p_ = i % nbuf
            sj = start_of(i)
            s_c = jnp.minimum(sj, wrap_lim)
            s8 = pl.multiple_of(jnp.minimum((s_c // 8) * 8, p - LF), 8)
            r0 = jnp.maximum(sj - wrap_lim, 0)
            r8 = pl.multiple_of((r0 // 8) * 8, 8)
            d = jnp.where(sj <= wrap_lim, s_c - s8, r0 - r8)
            main_cp = pltpu.make_async_copy(
                table_hbm.at[pl.ds(s8, LF)], bufs[p_], in_sems.at[p_]
            )
            edge_cp = pltpu.make_async_copy(
                edge_hbm.at[pl.ds(r8, LF)], bufs[p_], in_sems.at[p_]
            )

            @pl.when(sj <= wrap_lim)
            def _():
                main_cp.start()

            @pl.when(sj > wrap_lim)
            def _():
                edge_cp.start()

            return main_cp, d

        def scatter(i, d):
            p_ = i % nbuf
            return pltpu.make_async_copy(
                bufs[p_].at[pl.ds(d, L)], out_hbm.at[base + i], out_sems.at[p_]
            )

        depth = nbuf // 2
        in_h = {}
        out_h = {}
        for i in range(per_w + depth):
            if i < per_w:
                if i >= nbuf:
                    out_h[i - nbuf].wait()
                in_h[i] = gather(i)
            j = i - depth
            if 0 <= j < per_w:
                cp, d = in_h[j]
                cp.wait()
                out_h[j] = scatter(j, d)
                out_h[j].start()
        for i in range(max(0, per_w - nbuf), per_w):
            out_h[i].wait()

    return k(starts, table_flat, edge_flat)


def kernel(index, length, data):
    p = data.shape[0]
    batch = index.shape[0]
    starts = jnp.mod(index + (jnp.asarray(length, index.dtype) - L), p)
    edge = jnp.concatenate([data[p - L :], data[:L]], axis=0)
    return _sc_block_gather(starts.astype(jnp.int32), data, edge, batch, p)


# speedup vs baseline: 1.0477x; 1.0477x over previous
import functools

import jax
import jax.numpy as jnp
from jax import lax
from jax.experimental import pallas as pl
from jax.experimental.pallas import tpu as pltpu
from jax.experimental.pallas import tpu_sc as plsc

L = 200
C = 64


def _sc_block_gather(starts, table_flat, edge_flat, batch, p):
    num_workers = 32
    per_w = batch // num_workers
    blk = L * C
    nbuf = 4
    wrap_lim = p - L
    LF = L + 8
    mesh = plsc.VectorSubcoreMesh(core_axis_name="c", subcore_axis_name="s")

    @functools.partial(
        pl.kernel,
        out_type=jax.ShapeDtypeStruct((batch, L, C), jnp.float32),
        mesh=mesh,
        scratch_types=[
            pltpu.VMEM((per_w,), jnp.int32),
            pltpu.VMEM_SHARED((16, nbuf, LF, C), jnp.float32),
            pltpu.SemaphoreType.DMA((nbuf,)),
            pltpu.SemaphoreType.DMA((nbuf,)),
        ],
    )
    def k(starts_hbm, table_hbm, edge_hbm, out_hbm, idx_v, sbufs, in_sems, out_sems):
        sid = lax.axis_index("s")
        wid = sid * 2 + lax.axis_index("c")
        base = wid * per_w
        bufs = [sbufs.at[sid, q] for q in range(nbuf)]
        pltpu.sync_copy(starts_hbm.at[pl.ds(base, per_w)], idx_v)
        svecs = [idx_v[pl.ds(g * 16, 16)] for g in range(per_w // 16)]

        def start_of(i):
            return svecs[i // 16][i % 16]

        def gather(i):
            p_ = i % nbuf
            sj = start_of(i)
            s_c = jnp.minimum(sj, wrap_lim)
            s8 = pl.multiple_of(jnp.minimum((s_c // 8) * 8, p - LF), 8)
            r0 = jnp.maximum(sj - wrap_lim, 0)
            r8 = pl.multiple_of((r0 // 8) * 8, 8)
            d = jnp.where(sj <= wrap_lim, s_c - s8, r0 - r8)
            main_cp = pltpu.make_async_copy(
                table_hbm.at[pl.ds(s8, LF)], bufs[p_], in_sems.at[p_]
            )
            edge_cp = pltpu.make_async_copy(
                edge_hbm.at[pl.ds(r8, LF)], bufs[p_], in_sems.at[p_]
            )

            @pl.when(sj <= wrap_lim)
            def _():
                main_cp.start()

            @pl.when(sj > wrap_lim)
            def _():
                edge_cp.start()

            return main_cp, d

        def scatter(i, d):
            p_ = i % nbuf
            return pltpu.make_async_copy(
                sbufs.at[sid, p_, pl.ds(d, L)], out_hbm.at[base + i], out_sems.at[p_]
            )

        depth = nbuf // 2
        in_h = {}
        out_h = {}
        for i in range(per_w + depth):
            if i < per_w:
                if i >= nbuf:
                    out_h[i - nbuf].wait()
                in_h[i] = gather(i)
            j = i - depth
            if 0 <= j < per_w:
                cp, d = in_h[j]
                cp.wait()
                out_h[j] = scatter(j, d)
                out_h[j].start()
        for i in range(max(0, per_w - nbuf), per_w):
            out_h[i].wait()

    return k(starts, table_flat, edge_flat)


def kernel(index, length, data):
    p = data.shape[0]
    batch = index.shape[0]
    starts = jnp.mod(index + (jnp.asarray(length, index.dtype) - L), p)
    edge = jnp.concatenate([data[p - L :], data[:L]], axis=0)
    return _sc_block_gather(starts.astype(jnp.int32), data, edge, batch, p)
